# baseline (speedup 1.0000x reference)
import jax
import jax.numpy as jnp
from jax.experimental import pallas as pl
from jax.experimental.pallas import tpu as pltpu

_N = 5000
_NP = 5120
_NB = _NP // 128
_NMS_T = 0.5
_SCORE_T = 0.05
_K = 100
_KP = 104


def _dot(a, b):
    return jax.lax.dot_general(
        a, b, (((1,), (0,)), ((), ())), preferred_element_type=jnp.float32
    )


def _iou_mask(ax1, ay1, ax2, ay2, aar, bx1, by1, bx2, by2, bar):
    w = jnp.maximum(jnp.minimum(ax2, bx2) - jnp.maximum(ax1, bx1), 0.0)
    h = jnp.maximum(jnp.minimum(ay2, by2) - jnp.maximum(ay1, by1), 0.0)
    inter = w * h
    iou = inter / (aar + bar - inter + 1e-9)
    return (iou > _NMS_T).astype(jnp.float32)


def _nms_body(d_ref, idx_ref, sst_ref, out_ref, kall_ref, rank_ref):
    kall_ref[...] = jnp.zeros((_NB, 128), jnp.float32)

    it = jax.lax.broadcasted_iota(jnp.int32, (128, 128), 0)
    jt = jax.lax.broadcasted_iota(jnp.int32, (128, 128), 1)
    tri = (it < jt).astype(jnp.float32)

    def block_cols(e):
        ids = jnp.transpose(idx_ref[pl.ds(e, 1), :])
        jiota = jax.lax.broadcasted_iota(jnp.int32, (128, _NP), 1)
        onehot = (jiota == ids).astype(jnp.float32)
        return _dot(onehot, d_ref[...])

    def block_iou(a, bx1, by1, bx2, by2, bar):
        return _iou_mask(a[:, 0:1], a[:, 1:2], a[:, 2:3], a[:, 3:4],
                         a[:, 5:6], bx1, by1, bx2, by2, bar)

    def blk_body(carry):
        i, cnt, _ = carry
        bc = block_cols(i)
        bt = jnp.transpose(bc)
        bx1 = bt[0:1, :]
        by1 = bt[1:2, :]
        bx2 = bt[2:3, :]
        by2 = bt[3:4, :]
        bar = bt[5:6, :]

        def ext_body(e, counts):
            m = block_iou(block_cols(e), bx1, by1, bx2, by2, bar)
            ke = kall_ref[pl.ds(e, 1), :]
            return counts + _dot(ke, m)

        counts = jax.lax.fori_loop(
            0, i, ext_body, jnp.zeros((1, 128), jnp.float32)
        )
        free = (counts == 0.0).astype(jnp.float32)

        mu = block_iou(bc, bx1, by1, bx2, by2, bar) * tri

        def fp_cond(c):
            return c[1]

        def fp_body(c):
            k, _ = c
            kn = free * (_dot(k, mu) == 0.0).astype(jnp.float32)
            return kn, jnp.any(kn != k)

        k, _ = jax.lax.while_loop(fp_cond, fp_body, (free, True))
        ke = k * (bt[4:5, :] > _SCORE_T).astype(jnp.float32)
        kall_ref[pl.ds(i, 1), :] = ke

        cnt = cnt + jnp.sum(ke)
        nxt = i + 1
        cont = ((nxt < _NB) & (cnt < float(_K))
                & (sst_ref[jnp.minimum(nxt, _NB - 1)] > _SCORE_T))
        return nxt, cnt, cont

    i_end, _, _ = jax.lax.while_loop(
        lambda c: c[2], blk_body, (0, 0.0, sst_ref[0] > _SCORE_T)
    )

    elig = kall_ref[...]

    incl = (it <= jt).astype(jnp.float32)
    rowcum = _dot(elig, incl)
    rowsum = rowcum[:, 127:128]
    bi = jax.lax.broadcasted_iota(jnp.int32, (_NB, _NB), 0)
    bj = jax.lax.broadcasted_iota(jnp.int32, (_NB, _NB), 1)
    lstrict = (bj < bi).astype(jnp.float32)
    offs = _dot(lstrict, rowsum)
    rank_ref[...] = rowcum + offs - elig

    riota = jax.lax.broadcasted_iota(jnp.int32, (_KP, 1), 0).astype(
        jnp.float32
    )

    def out_body(i, acc):
        r = rank_ref[pl.ds(i, 1), :]
        e = kall_ref[pl.ds(i, 1), :]
        onehot = (riota == r).astype(jnp.float32) * e
        return acc + _dot(onehot, block_cols(i))

    out_ref[...] = jax.lax.fori_loop(
        0, i_end, out_body, jnp.zeros((_KP, 8), jnp.float32)
    )


@jax.jit
def kernel(boxes, scores):
    pad = _NP - _N
    zpad = jnp.zeros((pad,), jnp.float32)
    s0 = jnp.concatenate([scores, jnp.full((pad,), -1.0, jnp.float32)])
    x1 = jnp.concatenate([boxes[:, 0], zpad])
    y1 = jnp.concatenate([boxes[:, 1], zpad])
    x2 = jnp.concatenate([boxes[:, 2], zpad])
    y2 = jnp.concatenate([boxes[:, 3], zpad])
    area = (x2 - x1) * (y2 - y1)

    neg_s, idx = jax.lax.sort(
        (-s0, jax.lax.iota(jnp.int32, _NP)), dimension=0, num_keys=1
    )
    s = -neg_s

    d = jnp.stack(
        [x1, y1, x2, y2, s0, area, jnp.zeros_like(s0), jnp.zeros_like(s0)],
        axis=1,
    )
    sst = s[:: 128]

    out = pl.pallas_call(
        _nms_body,
        out_shape=jax.ShapeDtypeStruct((_KP, 8), jnp.float32),
        in_specs=[pl.BlockSpec(memory_space=pltpu.VMEM),
                  pl.BlockSpec(memory_space=pltpu.VMEM),
                  pl.BlockSpec(memory_space=pltpu.SMEM)],
        out_specs=pl.BlockSpec(memory_space=pltpu.VMEM),
        scratch_shapes=[pltpu.VMEM((_NB, 128), jnp.float32),
                        pltpu.VMEM((_NB, 128), jnp.float32)],
    )(d, idx.reshape(_NB, 128), sst)

    return out[:_K, :5]


# speedup vs baseline: 1.1990x; 1.1990x over previous
import jax
import jax.numpy as jnp
from jax.experimental import pallas as pl
from jax.experimental.pallas import tpu as pltpu

_N = 5000
_NP = 5120
_NB = _NP // 128
_NMS_T = 0.5
_SCORE_T = 0.05
_K = 100
_KP = 104


def _dot(a, b):
    return jax.lax.dot_general(
        a, b, (((1,), (0,)), ((), ())), preferred_element_type=jnp.float32
    )


def _iou_mask(ax1, ay1, ax2, ay2, aar, bx1, by1, bx2, by2, bar):
    w = jnp.maximum(jnp.minimum(ax2, bx2) - jnp.maximum(ax1, bx1), 0.0)
    h = jnp.maximum(jnp.minimum(ay2, by2) - jnp.maximum(ay1, by1), 0.0)
    inter = w * h
    iou = inter / (aar + bar - inter + 1e-9)
    return (iou > _NMS_T).astype(jnp.float32)


def _nms_body(dcol_ref, sst_ref, out_ref, kall_ref, rank_ref):
    kall_ref[...] = jnp.zeros((_NB, 128), jnp.float32)

    it = jax.lax.broadcasted_iota(jnp.int32, (128, 128), 0)
    jt = jax.lax.broadcasted_iota(jnp.int32, (128, 128), 1)
    tri = (it < jt).astype(jnp.float32)

    def block_cols(e):
        return jnp.transpose(dcol_ref[:, pl.ds(e * 128, 128)])

    def block_iou(a, bx1, by1, bx2, by2, bar):
        return _iou_mask(a[:, 0:1], a[:, 1:2], a[:, 2:3], a[:, 3:4],
                         a[:, 5:6], bx1, by1, bx2, by2, bar)

    def brow(c, i):
        return dcol_ref[c:c + 1, pl.ds(i * 128, 128)]

    def blk_body(carry):
        i, cnt, _ = carry
        bx1 = brow(0, i)
        by1 = brow(1, i)
        bx2 = brow(2, i)
        by2 = brow(3, i)
        bar = brow(5, i)

        def ext_body(e, counts):
            m = block_iou(block_cols(e), bx1, by1, bx2, by2, bar)
            ke = kall_ref[pl.ds(e, 1), :]
            return counts + _dot(ke, m)

        counts = jax.lax.fori_loop(
            0, i, ext_body, jnp.zeros((1, 128), jnp.float32)
        )
        free = (counts == 0.0).astype(jnp.float32)

        mu = block_iou(block_cols(i), bx1, by1, bx2, by2, bar) * tri

        def fp_cond(c):
            return c[1]

        def fp_body(c):
            k, _ = c
            kn = free * (_dot(k, mu) == 0.0).astype(jnp.float32)
            return kn, jnp.any(kn != k)

        k, _ = jax.lax.while_loop(fp_cond, fp_body, (free, True))
        ke = k * (brow(4, i) > _SCORE_T).astype(jnp.float32)
        kall_ref[pl.ds(i, 1), :] = ke

        cnt = cnt + jnp.sum(ke)
        nxt = i + 1
        cont = ((nxt < _NB) & (cnt < float(_K))
                & (sst_ref[jnp.minimum(nxt, _NB - 1)] > _SCORE_T))
        return nxt, cnt, cont

    i_end, _, _ = jax.lax.while_loop(
        lambda c: c[2], blk_body, (0, 0.0, sst_ref[0] > _SCORE_T)
    )

    elig = kall_ref[...]

    incl = (it <= jt).astype(jnp.float32)
    rowcum = _dot(elig, incl)
    rowsum = rowcum[:, 127:128]
    bi = jax.lax.broadcasted_iota(jnp.int32, (_NB, _NB), 0)
    bj = jax.lax.broadcasted_iota(jnp.int32, (_NB, _NB), 1)
    lstrict = (bj < bi).astype(jnp.float32)
    offs = _dot(lstrict, rowsum)
    rank_ref[...] = rowcum + offs - elig

    riota = jax.lax.broadcasted_iota(jnp.int32, (_KP, 1), 0).astype(
        jnp.float32
    )

    def out_body(i, acc):
        r = rank_ref[pl.ds(i, 1), :]
        e = kall_ref[pl.ds(i, 1), :]
        onehot = (riota == r).astype(jnp.float32) * e
        return acc + _dot(onehot, block_cols(i))

    out_ref[...] = jax.lax.fori_loop(
        0, i_end, out_body, jnp.zeros((_KP, 8), jnp.float32)
    )


@jax.jit
def kernel(boxes, scores):
    neg_s, x1, y1, x2, y2 = jax.lax.sort(
        (-scores, boxes[:, 0], boxes[:, 1], boxes[:, 2], boxes[:, 3]),
        dimension=0, num_keys=1,
    )
    s = -neg_s

    pad = _NP - _N
    zpad = jnp.zeros((pad,), jnp.float32)
    s = jnp.concatenate([s, jnp.full((pad,), -1.0, jnp.float32)])
    x1 = jnp.concatenate([x1, zpad])
    y1 = jnp.concatenate([y1, zpad])
    x2 = jnp.concatenate([x2, zpad])
    y2 = jnp.concatenate([y2, zpad])
    area = (x2 - x1) * (y2 - y1)

    dcol = jnp.stack(
        [x1, y1, x2, y2, s, area, jnp.zeros_like(s), jnp.zeros_like(s)],
        axis=0,
    )
    sst = s[:: 128]

    out = pl.pallas_call(
        _nms_body,
        out_shape=jax.ShapeDtypeStruct((_KP, 8), jnp.float32),
        in_specs=[pl.BlockSpec(memory_space=pltpu.VMEM),
                  pl.BlockSpec(memory_space=pltpu.SMEM)],
        out_specs=pl.BlockSpec(memory_space=pltpu.VMEM),
        scratch_shapes=[pltpu.VMEM((_NB, 128), jnp.float32),
                        pltpu.VMEM((_NB, 128), jnp.float32)],
    )(dcol, sst)

    return out[:_K, :5]
